# Initial kernel scaffold; baseline (speedup 1.0000x reference)
import jax
import jax.numpy as jnp
from jax.experimental import pallas as pl


def kernel(features, laplacianMat_indices, laplacianMat_values, selfLoop):
    raise NotImplementedError("write your pallas kernel here")


# speedup vs baseline: 6.6869x; 6.6869x over previous
import functools

import jax
import jax.numpy as jnp
from jax import lax
from jax.experimental import pallas as pl
from jax.experimental.pallas import tpu as pltpu
from jax.experimental.pallas import tpu_sc as plsc

N_NODES = 10000
N_EDGES = 320000
D_FEAT = 128
G = 128
NGROUPS = N_EDGES // G
NC = 2
NS = 16
NW = NC * NS
N_PAD = 10240
ROWS_PER_TILE = N_PAD // NS


def _sc_partials(features, col2, row2, val2, zeros):
    mesh = plsc.VectorSubcoreMesh(core_axis_name="c", subcore_axis_name="s")

    @functools.partial(
        pl.kernel,
        out_type=jax.ShapeDtypeStruct((NC, N_PAD, D_FEAT), jnp.float32),
        mesh=mesh,
        scratch_types=[
            pltpu.VMEM((G,), jnp.int32),
            pltpu.VMEM((G,), jnp.int32),
            pltpu.VMEM((G,), jnp.float32),
            pltpu.VMEM((G, D_FEAT), jnp.float32),
            pltpu.VMEM_SHARED((N_PAD, D_FEAT), jnp.float32),
            pltpu.SemaphoreType.DMA,
            pltpu.SemaphoreType.DMA,
            pltpu.SemaphoreType.DMA,
            pltpu.SemaphoreType.DMA,
            pltpu.SemaphoreType.DMA,
        ],
    )
    def k(feat_hbm, col_hbm, row_hbm, val_hbm, zero_hbm, out_hbm,
          cidx_v, ridx_v, val_v, rows_v, acc, s_c, s_r, s_v, s_g, s_s):
        c = lax.axis_index("c")
        s = lax.axis_index("s")
        wid = s * NC + c

        r0 = s * ROWS_PER_TILE
        pltpu.sync_copy(zero_hbm.at[pl.ds(r0, ROWS_PER_TILE)],
                        acc.at[pl.ds(r0, ROWS_PER_TILE)])
        plsc.subcore_barrier()

        ng = jnp.where(wid < NGROUPS % NW, NGROUPS // NW + 1, NGROUPS // NW)

        def group_body(k_i, _):
            g = wid + k_i * NW
            cp_c = pltpu.async_copy(col_hbm.at[g], cidx_v, s_c)
            cp_r = pltpu.async_copy(row_hbm.at[g], ridx_v, s_r)
            cp_v = pltpu.async_copy(val_hbm.at[g], val_v, s_v)
            cp_c.wait()
            pltpu.async_copy(feat_hbm.at[cidx_v], rows_v, s_g).wait()
            cp_v.wait()

            def scale_body(t, _):
                ve = val_v[pl.ds(16 * t, 16)]
                for l in range(16):
                    e = 16 * t + l
                    vv = jnp.full((16,), ve[l], jnp.float32)
                    for j in range(D_FEAT // 16):
                        sl = pl.ds(16 * j, 16)
                        rows_v[e, sl] = rows_v[e, sl] * vv
                return 0

            lax.fori_loop(0, G // 16, scale_body, 0)
            cp_r.wait()
            pltpu.async_copy(rows_v, acc.at[ridx_v], s_s, add=True).wait()
            return 0

        lax.fori_loop(0, ng, group_body, 0)

        plsc.subcore_barrier()
        pltpu.sync_copy(acc.at[pl.ds(r0, ROWS_PER_TILE)],
                        out_hbm.at[c, pl.ds(r0, ROWS_PER_TILE)])

    return k(features, col2, row2, val2, zeros)


def _combine_kernel(p_ref, o_ref):
    o_ref[...] = p_ref[0] + p_ref[1]


def _combine(partials):
    blk = 1000
    return pl.pallas_call(
        _combine_kernel,
        out_shape=jax.ShapeDtypeStruct((N_NODES, D_FEAT), jnp.float32),
        grid=(N_NODES // blk,),
        in_specs=[pl.BlockSpec((NC, blk, D_FEAT), lambda i: (0, i, 0))],
        out_specs=pl.BlockSpec((blk, D_FEAT), lambda i: (i, 0)),
    )(partials)


def kernel(features, laplacianMat_indices, laplacianMat_values, selfLoop):
    del selfLoop
    row2 = laplacianMat_indices[0].reshape(NGROUPS, G)
    col2 = laplacianMat_indices[1].reshape(NGROUPS, G)
    val2 = laplacianMat_values.reshape(NGROUPS, G)
    zeros = jnp.zeros((N_PAD, D_FEAT), jnp.float32)
    partials = _sc_partials(features, col2, row2, val2, zeros)
    return _combine(partials)
